# baseline (speedup 1.0000x reference)
import functools

import jax
import jax.numpy as jnp
from jax import lax
from jax.experimental import pallas as pl
from jax.experimental.pallas import tpu as pltpu
from jax.experimental.pallas import tpu_sc as plsc

N_NODES = 10000
N_EDGES = 320000
FEATS = 128

NC = 2
NS = 16
NW = NC * NS
EPW = N_EDGES // NW
CH = 80
NCH = EPW // CH
RPT = 624
CHR = 104
TAIL = N_NODES - NS * RPT
LG = FEATS // 16


def _mm_xw_kernel(x_ref, w_ref, o_ref):
    o_ref[...] = jnp.dot(x_ref[...], w_ref[...],
                         preferred_element_type=jnp.float32)


def _mm_fc_kernel(p_ref, w_ref, bg_ref, bf_ref, o_ref):
    h = p_ref[0] + p_ref[1] + bg_ref[...]
    o_ref[...] = lax.dot_general(
        h, w_ref[...], (((1,), (1,)), ((), ())),
        preferred_element_type=jnp.float32) + bf_ref[...]


def _sc_scatter_body(xw_hbm, src_hbm, dst_hbm, attr_hbm, out_hbm,
                     src_v, dst_v, attr_v, rows0, rows1, acc,
                     gsem0, gsem1, ssem0, ssem1):
    rows_v = rows0
    c = lax.axis_index("c")
    s = lax.axis_index("s")
    wid = s * NC + c

    zero16 = jnp.zeros((16,), jnp.float32)

    def zrow(i, carry):
        for j in range(LG):
            rows_v[i, pl.ds(16 * j, 16)] = zero16
        return carry

    lax.fori_loop(0, CH, zrow, 0)
    for k in range(7):
        pltpu.sync_copy(rows_v, acc.at[pl.ds(s * RPT + k * CH, CH)])
    pltpu.sync_copy(rows_v.at[pl.ds(0, 64)],
                    acc.at[pl.ds(s * RPT + 7 * CH, 64)])

    @pl.when(s == NS - 1)
    def _zero_tail():
        pltpu.sync_copy(rows_v.at[pl.ds(0, TAIL)],
                        acc.at[pl.ds(NS * RPT, TAIL)])

    plsc.subcore_barrier()

    base = wid * EPW
    pltpu.sync_copy(src_hbm.at[pl.ds(base, EPW)], src_v)
    pltpu.sync_copy(dst_hbm.at[pl.ds(base, EPW)], dst_v)
    pltpu.sync_copy(attr_hbm.at[pl.ds(base, EPW)], attr_v)

    bufs = (rows0, rows1)
    gsems = (gsem0, gsem1)
    ssems = (ssem0, ssem1)

    def _gather(k, b):
        return pltpu.async_copy(xw_hbm.at[src_v.at[pl.ds(k * CH, CH)]],
                                bufs[b], gsems[b])

    def _gather_wait(k, b):
        pltpu.make_async_copy(xw_hbm.at[src_v.at[pl.ds(k * CH, CH)]],
                              bufs[b], gsems[b]).wait()

    def _scatter(k, b):
        return pltpu.async_copy(bufs[b],
                                acc.at[dst_v.at[pl.ds(k * CH, CH)]],
                                ssems[b], add=True)

    def _scatter_wait(k, b):
        pltpu.make_async_copy(bufs[b],
                              acc.at[dst_v.at[pl.ds(k * CH, CH)]],
                              ssems[b]).wait()

    def _scale(k, b):
        buf = bufs[b]
        off = k * CH

        for g in range(CH // 16):
            a_vec = attr_v[pl.ds(off + g * 16, 16)]
            for i in range(16):
                a = a_vec[i]
                for j in range(LG):
                    buf[g * 16 + i, pl.ds(16 * j, 16)] = (
                        buf[g * 16 + i, pl.ds(16 * j, 16)] * a)

    _gather(0, 0)
    _gather_wait(0, 0)
    _gather(1, 1)
    _scale(0, 0)
    _scatter(0, 0)

    def pair(k2, carry):
        for half in range(2):
            k = k2 * 2 + 1 + half
            b = (1 + half) % 2
            _gather_wait(k, b)
            _scatter_wait(k - 1, b ^ 1)

            @pl.when(k + 1 < NCH)
            def _issue():
                _gather(k + 1, b ^ 1)

            _scale(k, b)
            _scatter(k, b)
        return carry

    lax.fori_loop(0, (NCH - 1) // 2, pair, 0)
    _scatter_wait(NCH - 1, 0)
    plsc.subcore_barrier()

    for k in range(RPT // CHR):
        start = s * RPT + k * CHR
        pltpu.sync_copy(acc.at[pl.ds(start, CHR)],
                        out_hbm.at[c, pl.ds(start, CHR)])

    @pl.when(s == NS - 1)
    def _copy_tail():
        pltpu.sync_copy(acc.at[pl.ds(NS * RPT, TAIL)],
                        out_hbm.at[c, pl.ds(NS * RPT, TAIL)])


_sc_scatter = functools.partial(
    pl.kernel,
    mesh=plsc.VectorSubcoreMesh(core_axis_name="c", subcore_axis_name="s"),
    out_type=jax.ShapeDtypeStruct((NC, N_NODES, FEATS), jnp.float32),
    scratch_types=[
        pltpu.VMEM((EPW,), jnp.int32),
        pltpu.VMEM((EPW,), jnp.int32),
        pltpu.VMEM((EPW,), jnp.float32),
        pltpu.VMEM((CH, FEATS), jnp.float32),
        pltpu.VMEM((CH, FEATS), jnp.float32),
        pltpu.VMEM_SHARED((N_NODES, FEATS), jnp.float32),
        pltpu.SemaphoreType.DMA,
        pltpu.SemaphoreType.DMA,
        pltpu.SemaphoreType.DMA,
        pltpu.SemaphoreType.DMA,
    ],
)(_sc_scatter_body)


def kernel(inputs, edge_index, edge_attr, W_gcn, b_gcn, W_fc, b_fc):
    src = edge_index[0].astype(jnp.int32)
    dst = edge_index[1].astype(jnp.int32)
    attr = edge_attr.astype(jnp.float32)

    blk = 1000
    grid = N_NODES // blk
    xw = pl.pallas_call(
        _mm_xw_kernel,
        grid=(grid,),
        in_specs=[
            pl.BlockSpec((blk, FEATS), lambda i: (i, 0)),
            pl.BlockSpec((FEATS, FEATS), lambda i: (0, 0)),
        ],
        out_specs=pl.BlockSpec((blk, FEATS), lambda i: (i, 0)),
        out_shape=jax.ShapeDtypeStruct((N_NODES, FEATS), jnp.float32),
    )(inputs, W_gcn)

    partials = _sc_scatter(xw, src, dst, attr)

    preds = pl.pallas_call(
        _mm_fc_kernel,
        grid=(grid,),
        in_specs=[
            pl.BlockSpec((NC, blk, FEATS), lambda i: (0, i, 0)),
            pl.BlockSpec((FEATS, FEATS), lambda i: (0, 0)),
            pl.BlockSpec((1, FEATS), lambda i: (0, 0)),
            pl.BlockSpec((1, FEATS), lambda i: (0, 0)),
        ],
        out_specs=pl.BlockSpec((blk, FEATS), lambda i: (i, 0)),
        out_shape=jax.ShapeDtypeStruct((N_NODES, FEATS), jnp.float32),
    )(partials, W_fc, b_gcn.reshape(1, FEATS), b_fc.reshape(1, FEATS))

    return preds


# speedup vs baseline: 1.1055x; 1.1055x over previous
import functools

import jax
import jax.numpy as jnp
from jax import lax
from jax.experimental import pallas as pl
from jax.experimental.pallas import tpu as pltpu
from jax.experimental.pallas import tpu_sc as plsc

N_NODES = 10000
N_EDGES = 320000
FEATS = 128

NC = 2
NS = 16
NW = NC * NS
EPW = N_EDGES // NW
CH = 128
EPH1 = 5120
NCH1 = EPH1 // CH
EPH2 = EPW - EPH1
NCH2 = EPH2 // CH
TCH = EPH2 - NCH2 * CH
TOFF = NCH2 * CH
RPT = 624
CHR = 104
TAIL = N_NODES - NS * RPT
LG = FEATS // 16


def _mm_xw_kernel(x_ref, w_ref, o_ref):
    o_ref[...] = jnp.dot(x_ref[...], w_ref[...],
                         preferred_element_type=jnp.float32)


def _mm_fc_kernel(p_ref, w_ref, bg_ref, bf_ref, o_ref):
    h = p_ref[0] + p_ref[1] + bg_ref[...]
    o_ref[...] = lax.dot_general(
        h, w_ref[...], (((1,), (1,)), ((), ())),
        preferred_element_type=jnp.float32) + bf_ref[...]


def _sc_scatter_body(xw_hbm, src_hbm, dst_hbm, attr_hbm, out_hbm,
                     src_v, dst_v, attr_v, rows0, rows1, acc,
                     gsem0, gsem1, ssem0, ssem1):
    rows_v = rows0
    c = lax.axis_index("c")
    s = lax.axis_index("s")
    wid = s * NC + c

    zero16 = jnp.zeros((16,), jnp.float32)

    def zrow(i, carry):
        for j in range(LG):
            rows_v[i, pl.ds(16 * j, 16)] = zero16
        return carry

    lax.fori_loop(0, CH, zrow, 0)
    for k in range(4):
        pltpu.sync_copy(rows_v, acc.at[pl.ds(s * RPT + k * CH, CH)])
    pltpu.sync_copy(rows_v.at[pl.ds(0, 112)],
                    acc.at[pl.ds(s * RPT + 4 * CH, 112)])

    @pl.when(s == NS - 1)
    def _zero_tail():
        pltpu.sync_copy(rows_v.at[pl.ds(0, TAIL)],
                        acc.at[pl.ds(NS * RPT, TAIL)])

    plsc.subcore_barrier()

    bufs = (rows0, rows1)
    gsems = (gsem0, gsem1)
    ssems = (ssem0, ssem1)

    def _gather(k, b):
        return pltpu.async_copy(xw_hbm.at[src_v.at[pl.ds(k * CH, CH)]],
                                bufs[b], gsems[b])

    def _gather_wait(k, b):
        pltpu.make_async_copy(xw_hbm.at[src_v.at[pl.ds(k * CH, CH)]],
                              bufs[b], gsems[b]).wait()

    def _scatter(k, b):
        return pltpu.async_copy(bufs[b],
                                acc.at[dst_v.at[pl.ds(k * CH, CH)]],
                                ssems[b], add=True)

    def _scatter_wait(k, b):
        pltpu.make_async_copy(bufs[b],
                              acc.at[dst_v.at[pl.ds(k * CH, CH)]],
                              ssems[b]).wait()

    def _scale(k, b):
        buf = bufs[b]
        off = k * CH

        def scale16(g, inner):
            a_vec = attr_v[pl.ds(off + g * 16, 16)]
            for i in range(16):
                a = a_vec[i]
                for j in range(LG):
                    buf[g * 16 + i, pl.ds(16 * j, 16)] = (
                        buf[g * 16 + i, pl.ds(16 * j, 16)] * a)
            return inner

        lax.fori_loop(0, CH // 16, scale16, 0)

    def _tail_gather():
        return pltpu.async_copy(xw_hbm.at[src_v.at[pl.ds(TOFF, TCH)]],
                                bufs[0].at[pl.ds(0, TCH)], gsems[0])

    def _tail_gather_wait():
        pltpu.make_async_copy(xw_hbm.at[src_v.at[pl.ds(TOFF, TCH)]],
                              bufs[0].at[pl.ds(0, TCH)], gsems[0]).wait()

    def _tail_scale():
        a_vec = attr_v[pl.ds(TOFF, TCH)]
        for i in range(TCH):
            a = a_vec[i]
            for j in range(LG):
                bufs[0][i, pl.ds(16 * j, 16)] = (
                    bufs[0][i, pl.ds(16 * j, 16)] * a)

    def _tail_scatter():
        return pltpu.async_copy(bufs[0].at[pl.ds(0, TCH)],
                                acc.at[dst_v.at[pl.ds(TOFF, TCH)]],
                                ssems[0], add=True)

    def _tail_scatter_wait():
        pltpu.make_async_copy(bufs[0].at[pl.ds(0, TCH)],
                              acc.at[dst_v.at[pl.ds(TOFF, TCH)]],
                              ssems[0]).wait()

    base = wid * EPW
    for p, (poff, pedges, nch) in enumerate(
            ((0, EPH1, NCH1), (EPH1, EPH2, NCH2))):
        pltpu.sync_copy(src_hbm.at[pl.ds(base + poff, pedges)],
                        src_v.at[pl.ds(0, pedges)])
        pltpu.sync_copy(dst_hbm.at[pl.ds(base + poff, pedges)],
                        dst_v.at[pl.ds(0, pedges)])
        pltpu.sync_copy(attr_hbm.at[pl.ds(base + poff, pedges)],
                        attr_v.at[pl.ds(0, pedges)])

        _gather(0, 0)
        _gather_wait(0, 0)
        _gather(1, 1)
        _scale(0, 0)
        _scatter(0, 0)

        def pair(k2, carry, nch=nch):
            for half in range(2):
                k = k2 * 2 + 1 + half
                b = (1 + half) % 2
                _gather_wait(k, b)
                _scatter_wait(k - 1, b ^ 1)

                @pl.when(k + 1 < nch)
                def _issue():
                    _gather(k + 1, b ^ 1)

                _scale(k, b)
                _scatter(k, b)
            return carry

        lax.fori_loop(0, (nch - 2) // 2, pair, 0)

        _gather_wait(nch - 1, 1)
        _scatter_wait(nch - 2, 0)
        if p == 1:
            _tail_gather()
        _scale(nch - 1, 1)
        _scatter(nch - 1, 1)
        if p == 1:
            _tail_gather_wait()
            _scatter_wait(nch - 1, 1)
            _tail_scale()
            _tail_scatter()
            _tail_scatter_wait()
        else:
            _scatter_wait(nch - 1, 1)

    plsc.subcore_barrier()

    for k in range(RPT // CHR):
        start = s * RPT + k * CHR
        pltpu.sync_copy(acc.at[pl.ds(start, CHR)],
                        out_hbm.at[c, pl.ds(start, CHR)])

    @pl.when(s == NS - 1)
    def _copy_tail():
        pltpu.sync_copy(acc.at[pl.ds(NS * RPT, TAIL)],
                        out_hbm.at[c, pl.ds(NS * RPT, TAIL)])


_sc_scatter = functools.partial(
    pl.kernel,
    mesh=plsc.VectorSubcoreMesh(core_axis_name="c", subcore_axis_name="s"),
    out_type=jax.ShapeDtypeStruct((NC, N_NODES, FEATS), jnp.float32),
    scratch_types=[
        pltpu.VMEM((EPH1,), jnp.int32),
        pltpu.VMEM((EPH1,), jnp.int32),
        pltpu.VMEM((EPH1,), jnp.float32),
        pltpu.VMEM((CH, FEATS), jnp.float32),
        pltpu.VMEM((CH, FEATS), jnp.float32),
        pltpu.VMEM_SHARED((N_NODES, FEATS), jnp.float32),
        pltpu.SemaphoreType.DMA,
        pltpu.SemaphoreType.DMA,
        pltpu.SemaphoreType.DMA,
        pltpu.SemaphoreType.DMA,
    ],
)(_sc_scatter_body)


def kernel(inputs, edge_index, edge_attr, W_gcn, b_gcn, W_fc, b_fc):
    src = edge_index[0].astype(jnp.int32)
    dst = edge_index[1].astype(jnp.int32)
    attr = edge_attr.astype(jnp.float32)

    blk = 1000
    grid = N_NODES // blk
    xw = pl.pallas_call(
        _mm_xw_kernel,
        grid=(grid,),
        in_specs=[
            pl.BlockSpec((blk, FEATS), lambda i: (i, 0)),
            pl.BlockSpec((FEATS, FEATS), lambda i: (0, 0)),
        ],
        out_specs=pl.BlockSpec((blk, FEATS), lambda i: (i, 0)),
        out_shape=jax.ShapeDtypeStruct((N_NODES, FEATS), jnp.float32),
    )(inputs, W_gcn)

    partials = _sc_scatter(xw, src, dst, attr)

    preds = pl.pallas_call(
        _mm_fc_kernel,
        grid=(grid,),
        in_specs=[
            pl.BlockSpec((NC, blk, FEATS), lambda i: (0, i, 0)),
            pl.BlockSpec((FEATS, FEATS), lambda i: (0, 0)),
            pl.BlockSpec((1, FEATS), lambda i: (0, 0)),
            pl.BlockSpec((1, FEATS), lambda i: (0, 0)),
        ],
        out_specs=pl.BlockSpec((blk, FEATS), lambda i: (i, 0)),
        out_shape=jax.ShapeDtypeStruct((N_NODES, FEATS), jnp.float32),
    )(partials, W_fc, b_gcn.reshape(1, FEATS), b_fc.reshape(1, FEATS))

    return preds
